# baseline (speedup 1.0000x reference)
import functools

import numpy as np
import jax
import jax.numpy as jnp
from jax import lax
from jax.experimental import pallas as pl
from jax.experimental.pallas import tpu as pltpu
from jax.experimental.pallas import tpu_sc as plsc

B, T, P, N = 4096, 1000, 20, 50
PN = P * N
NC, NS = 2, 16
NW = NC * NS
ROWS_W = B // NW
RB = 16
NBLK = ROWS_W // RB
MB = RB * P
NFULL = PN // 16
TAIL0 = PN - 16
NTAILV = PN - NFULL * 16

_j2 = np.arange(MB)
_MRI = (_j2 // P).astype(np.int32)

_mesh = plsc.VectorSubcoreMesh(core_axis_name="c", subcore_axis_name="s")


@functools.partial(
    pl.kernel,
    out_type=jax.ShapeDtypeStruct((NW, 64), jnp.float32),
    mesh=_mesh,
    compiler_params=pltpu.CompilerParams(needs_layout_passes=False),
    scratch_types=[
        pltpu.VMEM((RB, T), jnp.float32),
        pltpu.VMEM((RB, T), jnp.float32),
        pltpu.VMEM((RB * PN,), jnp.int32),
        pltpu.VMEM((RB * PN,), jnp.int32),
        pltpu.VMEM((RB, P), jnp.int32),
        pltpu.VMEM((RB, P), jnp.int32),
        pltpu.VMEM((RB, P), jnp.int32),
        pltpu.VMEM((RB, P), jnp.int32),
        pltpu.VMEM((MB,), jnp.float32),
        pltpu.VMEM((16,), jnp.float32),
        pltpu.VMEM((64,), jnp.float32),
        pltpu.SemaphoreType.DMA,
        pltpu.SemaphoreType.DMA,
    ],
)
def _hinge_sc(scores_hbm, neg1d_hbm, pos_hbm, lev_hbm, marg_hbm,
              out_hbm,
              sc0, sc1, ng0, ng1, po0, po1, lv0, lv1,
              m_v, marg_v, out_v, sem0, sem1):
    wid = lax.axis_index("s") * NC + lax.axis_index("c")
    pltpu.sync_copy(marg_hbm, marg_v)
    row0w = wid * ROWS_W
    iota16 = lax.iota(jnp.int32, 16)
    bufs = ((sc0, ng0, po0, lv0, sem0), (sc1, ng1, po1, lv1, sem1))
    tail_keep = iota16 >= (16 - NTAILV)

    def issue(slot, bk):
        sc, ng, po, lv, sem = bufs[slot]
        row0 = row0w + bk * RB
        pltpu.async_copy(scores_hbm.at[pl.ds(row0, RB)], sc, sem)
        pltpu.async_copy(neg1d_hbm.at[pl.ds(row0 * PN, RB * PN)], ng, sem)
        pltpu.async_copy(pos_hbm.at[pl.ds(row0, RB)], po, sem)
        pltpu.async_copy(lev_hbm.at[pl.ds(row0, RB)], lv, sem)

    def wait_slot(slot):
        sc, ng, po, lv, sem = bufs[slot]
        pltpu.make_async_copy(scores_hbm.at[pl.ds(0, RB)], sc, sem).wait()
        pltpu.make_async_copy(neg1d_hbm.at[pl.ds(0, RB * PN)], ng, sem).wait()
        pltpu.make_async_copy(pos_hbm.at[pl.ds(0, RB)], po, sem).wait()
        pltpu.make_async_copy(lev_hbm.at[pl.ds(0, RB)], lv, sem).wait()

    def compute_block(slot, carry):
        sc, ng, po, lv, _ = bufs[slot]

        @plsc.parallel_loop(0, MB // 16, unroll=4)
        def mloop(i):
            off = i * 16
            j = iota16 + jnp.full((16,), off, jnp.int32)
            mri = j // P
            mpi = j - mri * P
            pos = plsc.load_gather(po, [mri, mpi])
            lev = plsc.load_gather(lv, [mri, mpi])
            safe_pos = jnp.where(pos == -1, 0, pos)
            psc = plsc.load_gather(sc, [mri, safe_pos])
            mg = plsc.load_gather(marg_v, [lev])
            m_v[pl.ds(off, 16)] = mg - psc

        def rowloop(r, cr):
            a = list(cr)
            rsplat = jnp.full((16,), r, jnp.int32)
            rp = r * P
            rbase = r * PN
            msp = [plsc.load_gather(m_v, [jnp.full((16,), rp + p, jnp.int32)])
                   for p in range(P)]

            def chunk(k, off, msel, extra_mask, a):
                neg = ng[pl.ds(rbase + off, 16)]
                msk = neg != -1
                if extra_mask is not None:
                    msk = msk & extra_mask
                safe_neg = jnp.where(msk, neg, 0)
                nsc = plsc.load_gather(sc, [rsplat, safe_neg])
                v = jnp.maximum(msel + nsc, 0.0)
                s = k % 4
                a[s] = a[s] + jnp.where(msk, v, 0.0)
                a[4 + s] = a[4 + s] + jnp.where(msk, 1, 0)
                return a

            for k in range(NFULL):
                off = k * 16
                p_lo = off // N
                p_hi = (off + 15) // N
                if p_lo == p_hi:
                    msel = msp[p_lo]
                else:
                    b = N * p_hi - off
                    msel = jnp.where(iota16 >= b, msp[p_hi], msp[p_lo])
                a = chunk(k, off, msel, None, a)
            a = chunk(NFULL, TAIL0, msp[P - 1], tail_keep, a)
            return tuple(a)

        return lax.fori_loop(0, RB, rowloop, carry)

    issue(0, 0)
    issue(1, 1)
    zf = jnp.zeros((16,), jnp.float32)
    zi = jnp.zeros((16,), jnp.int32)
    acc = (zf, zf, zf, zf, zi, zi, zi, zi)

    def blockpair(t, carry):
        wait_slot(0)
        carry = compute_block(0, carry)

        @pl.when(t < NBLK // 2 - 1)
        def _():
            issue(0, 2 * t + 2)

        wait_slot(1)
        carry = compute_block(1, carry)

        @pl.when(t < NBLK // 2 - 1)
        def _():
            issue(1, 2 * t + 3)

        return carry

    acc = lax.fori_loop(0, NBLK // 2, blockpair, acc)

    out_v[pl.ds(0, 16)] = acc[0] + acc[1]
    out_v[pl.ds(16, 16)] = acc[2] + acc[3]
    out_v[pl.ds(32, 16)] = (acc[4] + acc[5]).astype(jnp.float32)
    out_v[pl.ds(48, 16)] = (acc[6] + acc[7]).astype(jnp.float32)
    pltpu.sync_copy(out_v, out_hbm.at[wid])


def kernel(scores, pos_type_ids, neg_type_ids, levels, margin_ratio, margins):
    marg = (margins * margin_ratio).astype(jnp.float32)
    parts = _hinge_sc(
        scores,
        neg_type_ids.reshape(-1),
        pos_type_ids,
        levels,
        marg,
    )
    pr = parts.reshape(NW, 2, 32)
    return pr[:, 0].sum() / pr[:, 1].sum()


# speedup vs baseline: 1.1382x; 1.1382x over previous
import functools

import numpy as np
import jax
import jax.numpy as jnp
from jax import lax
from jax.experimental import pallas as pl
from jax.experimental.pallas import tpu as pltpu
from jax.experimental.pallas import tpu_sc as plsc

B, T, P, N = 4096, 1000, 20, 50
PN = P * N
NC, NS = 2, 16
NW = NC * NS
ROWS_W = B // NW
RB = 16
NBLK = ROWS_W // RB
MB = RB * P
NFULL = PN // 16
TAIL0 = PN - 16
NTAILV = PN - NFULL * 16

_j2 = np.arange(MB)
_MRI = (_j2 // P).astype(np.int32)

_mesh = plsc.VectorSubcoreMesh(core_axis_name="c", subcore_axis_name="s")


@functools.partial(
    pl.kernel,
    out_type=jax.ShapeDtypeStruct((NW, 64), jnp.float32),
    mesh=_mesh,
    compiler_params=pltpu.CompilerParams(needs_layout_passes=False),
    scratch_types=[
        pltpu.VMEM((RB, T), jnp.float32),
        pltpu.VMEM((RB, T), jnp.float32),
        pltpu.VMEM((RB, PN), jnp.int32),
        pltpu.VMEM((RB, PN), jnp.int32),
        pltpu.VMEM((RB, P), jnp.int32),
        pltpu.VMEM((RB, P), jnp.int32),
        pltpu.VMEM((RB, P), jnp.int32),
        pltpu.VMEM((RB, P), jnp.int32),
        pltpu.VMEM((MB,), jnp.float32),
        pltpu.VMEM((16,), jnp.float32),
        pltpu.VMEM((64,), jnp.float32),
        pltpu.SemaphoreType.DMA,
        pltpu.SemaphoreType.DMA,
    ],
)
def _hinge_sc(scores_hbm, neg2d_hbm, pos_hbm, lev_hbm, marg_hbm,
              out_hbm,
              sc0, sc1, ng0, ng1, po0, po1, lv0, lv1,
              m_v, marg_v, out_v, sem0, sem1):
    wid = lax.axis_index("s") * NC + lax.axis_index("c")
    pltpu.sync_copy(marg_hbm, marg_v)
    row0w = wid * ROWS_W
    iota16 = lax.iota(jnp.int32, 16)
    bufs = ((sc0, ng0, po0, lv0, sem0), (sc1, ng1, po1, lv1, sem1))
    tail_keep = iota16 >= (16 - NTAILV)

    def issue(slot, bk):
        sc, ng, po, lv, sem = bufs[slot]
        row0 = row0w + bk * RB
        pltpu.async_copy(scores_hbm.at[pl.ds(row0, RB)], sc, sem)
        pltpu.async_copy(neg2d_hbm.at[pl.ds(row0, RB)], ng, sem)
        pltpu.async_copy(pos_hbm.at[pl.ds(row0, RB)], po, sem)
        pltpu.async_copy(lev_hbm.at[pl.ds(row0, RB)], lv, sem)

    def wait_slot(slot):
        sc, ng, po, lv, sem = bufs[slot]
        pltpu.make_async_copy(scores_hbm.at[pl.ds(0, RB)], sc, sem).wait()
        pltpu.make_async_copy(neg2d_hbm.at[pl.ds(0, RB)], ng, sem).wait()
        pltpu.make_async_copy(pos_hbm.at[pl.ds(0, RB)], po, sem).wait()
        pltpu.make_async_copy(lev_hbm.at[pl.ds(0, RB)], lv, sem).wait()

    def compute_block(slot, carry):
        sc, ng, po, lv, _ = bufs[slot]

        @plsc.parallel_loop(0, MB // 16, unroll=4)
        def mloop(i):
            off = i * 16
            j = iota16 + jnp.full((16,), off, jnp.int32)
            mri = j // P
            mpi = j - mri * P
            pos = plsc.load_gather(po, [mri, mpi])
            lev = plsc.load_gather(lv, [mri, mpi])
            safe_pos = jnp.where(pos == -1, 0, pos)
            psc = plsc.load_gather(sc, [mri, safe_pos])
            mg = plsc.load_gather(marg_v, [lev])
            m_v[pl.ds(off, 16)] = mg - psc

        def rowloop(r, cr):
            a = list(cr)
            rsplat = jnp.full((16,), r, jnp.int32)
            rp = r * P
            msp = [plsc.load_gather(m_v, [jnp.full((16,), rp + p, jnp.int32)])
                   for p in range(P)]

            def chunk(k, off, msel, extra_mask, a):
                neg = ng[r, pl.ds(off, 16)]
                msk = neg != -1
                if extra_mask is not None:
                    msk = msk & extra_mask
                safe_neg = jnp.where(msk, neg, 0)
                nsc = plsc.load_gather(sc, [rsplat, safe_neg])
                v = jnp.maximum(msel + nsc, 0.0)
                s = k % 4
                a[s] = a[s] + jnp.where(msk, v, 0.0)
                a[4 + s] = a[4 + s] + plsc.all_reduce_population_count(msk)
                return a

            for k in range(NFULL):
                off = k * 16
                p_lo = off // N
                p_hi = (off + 15) // N
                if p_lo == p_hi:
                    msel = msp[p_lo]
                else:
                    b = N * p_hi - off
                    msel = jnp.where(iota16 >= b, msp[p_hi], msp[p_lo])
                a = chunk(k, off, msel, None, a)
            a = chunk(NFULL, TAIL0, msp[P - 1], tail_keep, a)
            return tuple(a)

        return lax.fori_loop(0, RB, rowloop, carry)

    issue(0, 0)
    issue(1, 1)
    zf = jnp.zeros((16,), jnp.float32)
    zi = jnp.zeros((16,), jnp.int32)
    acc = (zf, zf, zf, zf, zi, zi, zi, zi)

    def blockpair(t, carry):
        wait_slot(0)
        carry = compute_block(0, carry)

        @pl.when(t < NBLK // 2 - 1)
        def _():
            issue(0, 2 * t + 2)

        wait_slot(1)
        carry = compute_block(1, carry)

        @pl.when(t < NBLK // 2 - 1)
        def _():
            issue(1, 2 * t + 3)

        return carry

    acc = lax.fori_loop(0, NBLK // 2, blockpair, acc)

    out_v[pl.ds(0, 16)] = acc[0] + acc[1]
    out_v[pl.ds(16, 16)] = acc[2] + acc[3]
    out_v[pl.ds(32, 16)] = (acc[4] + acc[5]).astype(jnp.float32)
    out_v[pl.ds(48, 16)] = (acc[6] + acc[7]).astype(jnp.float32)
    pltpu.sync_copy(out_v, out_hbm.at[wid])


def kernel(scores, pos_type_ids, neg_type_ids, levels, margin_ratio, margins):
    marg = (margins * margin_ratio).astype(jnp.float32)
    parts = _hinge_sc(
        scores,
        neg_type_ids.reshape(B, PN),
        pos_type_ids,
        levels,
        marg,
    )
    pr = parts.reshape(NW, 2, 32)
    return pr[:, 0].sum() / (pr[:, 1].sum() / 16.0)
